# baseline (speedup 1.0000x reference)
import functools

import jax
import jax.numpy as jnp
from jax import lax
from jax.experimental import pallas as pl
from jax.experimental.pallas import tpu as pltpu
from jax.experimental.pallas import tpu_sc as plsc

NC = 2
NS = 16
NW = NC * NS
BATCH = 16384
BPW = BATCH // NW
NCHUNK = 4
CHUNK = BPW // NCHUNK
D = 50
NG = BPW // 16


def _collapse_body(W1r, b1r, W2r, b2r, W3r, b3r, outr):
    w32 = jnp.dot(W3r[...], W2r[...], preferred_element_type=jnp.float32)
    w100 = jnp.dot(w32, W1r[...], preferred_element_type=jnp.float32)
    c = jnp.sum(w32 * b1r[...]) + jnp.sum(W3r[...] * b2r[...]) + b3r[0, 0]
    outr[...] = jnp.concatenate(
        [w100, jnp.reshape(c, (1, 1)), jnp.zeros((1, 27), jnp.float32)], axis=1)


def _collapse(W1, b1, W2, b2, W3, b3):
    return pl.pallas_call(
        _collapse_body,
        out_shape=jax.ShapeDtypeStruct((1, 128), jnp.float32),
    )(W1, b1.reshape(1, 40), W2, b2.reshape(1, 20), W3, b3.reshape(1, 1))


def _sc_body(uidx_hbm, midx_hbm, utab_hbm, mtab_hbm, w_hbm, out_hbm,
             uidx_v, midx_v, urows_v, mrows_v, w_v, tbuf_v, out_v, sem):
    wid = lax.axis_index("s") * NC + lax.axis_index("c")

    pltpu.sync_copy(uidx_hbm.at[wid], uidx_v)
    pltpu.sync_copy(midx_hbm.at[wid], midx_v)
    pltpu.sync_copy(w_hbm, w_v)

    copies = []
    for j in range(NCHUNK):
        copies.append(pltpu.async_copy(
            utab_hbm.at[uidx_v.at[j]], urows_v.at[pl.ds(j * CHUNK, CHUNK)], sem))
        copies.append(pltpu.async_copy(
            mtab_hbm.at[midx_v.at[j]], mrows_v.at[pl.ds(j * CHUNK, CHUNK)], sem))
    for cp in copies:
        cp.wait()

    wu0 = w_v[0, :]
    wu1 = w_v[1, :]
    wu2 = w_v[2, :]
    wu3 = w_v[3, :]
    wm0 = w_v[4, :]
    wm1 = w_v[5, :]
    wm2 = w_v[6, :]
    wm3 = w_v[7, :]
    cvec = w_v[8, :]
    lanebase = lax.iota(jnp.int32, 16) * 16

    def group(g, carry):
        for j in range(16):
            row = g * 16 + j
            acc = urows_v[row, pl.ds(0, 16)] * wu0
            acc = acc + urows_v[row, pl.ds(16, 16)] * wu1
            acc = acc + urows_v[row, pl.ds(32, 16)] * wu2
            acc = acc + urows_v[row, pl.ds(34, 16)] * wu3
            acc = acc + mrows_v[row, pl.ds(0, 16)] * wm0
            acc = acc + mrows_v[row, pl.ds(16, 16)] * wm1
            acc = acc + mrows_v[row, pl.ds(32, 16)] * wm2
            acc = acc + mrows_v[row, pl.ds(34, 16)] * wm3
            tbuf_v[pl.ds(j * 16, 16)] = acc
        tot = cvec
        for f in range(16):
            tot = tot + plsc.load_gather(tbuf_v, [lanebase + f])
        out_v[pl.ds(g * 16, 16)] = tot
        return carry

    lax.fori_loop(0, NG, group, 0)
    pltpu.sync_copy(out_v, out_hbm.at[wid])


_sc_kernel = functools.partial(
    pl.kernel,
    out_type=jax.ShapeDtypeStruct((NW, BPW), jnp.float32),
    mesh=plsc.VectorSubcoreMesh(core_axis_name="c", subcore_axis_name="s",
                                num_cores=NC, num_subcores=NS),
    compiler_params=pltpu.CompilerParams(needs_layout_passes=False,
                                         use_tc_tiling_on_sc=False),
    scratch_types=[
        pltpu.VMEM((NCHUNK, CHUNK), jnp.int32),
        pltpu.VMEM((NCHUNK, CHUNK), jnp.int32),
        pltpu.VMEM((BPW, D), jnp.float32),
        pltpu.VMEM((BPW, D), jnp.float32),
        pltpu.VMEM((9, 16), jnp.float32),
        pltpu.VMEM((256,), jnp.float32),
        pltpu.VMEM((BPW,), jnp.float32),
        pltpu.SemaphoreType.DMA,
    ],
)(_sc_body)


def kernel(user, movie, user_table, movie_table, W1, b1, W2, b2, W3, b3):
    wf = _collapse(W1, b1, W2, b2, W3, b3)[0]
    tailmask = jnp.arange(16) >= 14
    zero16 = jnp.zeros((16,), jnp.float32)
    wrows = jnp.stack([
        wf[0:16], wf[16:32], wf[32:48],
        jnp.where(tailmask, wf[34:50], zero16),
        wf[50:66], wf[66:82], wf[82:98],
        jnp.where(tailmask, wf[84:100], zero16),
        jnp.full((16,), wf[100]),
    ])
    uidx = user.astype(jnp.int32).reshape(NW, NCHUNK, CHUNK)
    midx = movie.astype(jnp.int32).reshape(NW, NCHUNK, CHUNK)
    out = _sc_kernel(uidx, midx, user_table, movie_table, wrows)
    return out.reshape(BATCH, 1)


# speedup vs baseline: 5.4337x; 5.4337x over previous
import functools

import jax
import jax.numpy as jnp
from jax import lax
from jax.experimental import pallas as pl
from jax.experimental.pallas import tpu as pltpu
from jax.experimental.pallas import tpu_sc as plsc

NC = 2
NS = 16
NW = NC * NS
BATCH = 16384
BPW = BATCH // NW
NCHUNK = 4
CHUNK = BPW // NCHUNK
BLK = 4096


def _collapse_body(W1r, b1r, W2r, b2r, W3r, b3r, outr):
    w32 = jnp.dot(W3r[...], W2r[...], preferred_element_type=jnp.float32,
                  precision=lax.Precision.HIGHEST)
    w100 = jnp.dot(w32, W1r[...], preferred_element_type=jnp.float32,
                   precision=lax.Precision.HIGHEST)
    c = jnp.sum(w32 * b1r[...]) + jnp.sum(W3r[...] * b2r[...]) + b3r[0, 0]
    outr[...] = jnp.concatenate(
        [w100, jnp.reshape(c, (1, 1)), jnp.zeros((1, 27), jnp.float32)], axis=1)


def _collapse(W1, b1, W2, b2, W3, b3):
    return pl.pallas_call(
        _collapse_body,
        out_shape=jax.ShapeDtypeStruct((1, 128), jnp.float32),
    )(W1, b1.reshape(1, 40), W2, b2.reshape(1, 20), W3, b3.reshape(1, 1))


def _matvec_body(w_ref, tab_ref, bias_ref, out_ref):
    out_ref[...] = lax.dot_general(
        w_ref[...], tab_ref[...], (((1,), (0,)), ((), ())),
        precision=lax.Precision.HIGHEST,
        preferred_element_type=jnp.float32) + bias_ref[0, 0]


def _matvec(tab_t, w, bias):
    n = tab_t.shape[1]
    grid = (n + BLK - 1) // BLK
    out = pl.pallas_call(
        _matvec_body,
        grid=(grid,),
        in_specs=[
            pl.BlockSpec((1, 50), lambda i: (0, 0)),
            pl.BlockSpec((50, BLK), lambda i: (0, i)),
            pl.BlockSpec((1, 1), lambda i: (0, 0)),
        ],
        out_specs=pl.BlockSpec((1, BLK), lambda i: (0, i)),
        out_shape=jax.ShapeDtypeStruct((1, n), jnp.float32),
    )(w, tab_t, bias)
    return out.reshape(n)


def _sc_body(uidx_hbm, midx_hbm, udot_hbm, mdot_hbm, out_hbm,
             uidx_v, midx_v, uval_v, mval_v, sem):
    wid = lax.axis_index("s") * NC + lax.axis_index("c")
    for j in range(NCHUNK):
        pltpu.sync_copy(uidx_hbm.at[wid, pl.ds(j * CHUNK, CHUNK)], uidx_v.at[j])
        pltpu.sync_copy(midx_hbm.at[wid, pl.ds(j * CHUNK, CHUNK)], midx_v.at[j])
    copies = []
    for j in range(NCHUNK):
        copies.append(pltpu.async_copy(udot_hbm.at[uidx_v.at[j]], uval_v.at[j], sem))
        copies.append(pltpu.async_copy(mdot_hbm.at[midx_v.at[j]], mval_v.at[j], sem))
    for cp in copies:
        cp.wait()
    for j in range(NCHUNK):
        for v in range(CHUNK // 16):
            s = (uval_v[j, pl.ds(v * 16, 16)] + mval_v[j, pl.ds(v * 16, 16)])
            uval_v[j, pl.ds(v * 16, 16)] = s
        pltpu.sync_copy(uval_v.at[j], out_hbm.at[wid, pl.ds(j * CHUNK, CHUNK)])


_sc_kernel = functools.partial(
    pl.kernel,
    out_type=jax.ShapeDtypeStruct((NW, BPW), jnp.float32),
    mesh=plsc.VectorSubcoreMesh(core_axis_name="c", subcore_axis_name="s",
                                num_cores=NC, num_subcores=NS),
    compiler_params=pltpu.CompilerParams(needs_layout_passes=False,
                                         use_tc_tiling_on_sc=False),
    scratch_types=[
        pltpu.VMEM((NCHUNK, CHUNK), jnp.int32),
        pltpu.VMEM((NCHUNK, CHUNK), jnp.int32),
        pltpu.VMEM((NCHUNK, CHUNK), jnp.float32),
        pltpu.VMEM((NCHUNK, CHUNK), jnp.float32),
        pltpu.SemaphoreType.DMA,
    ],
)(_sc_body)


def kernel(user, movie, user_table, movie_table, W1, b1, W2, b2, W3, b3):
    wf = _collapse(W1, b1, W2, b2, W3, b3)
    zero = jnp.zeros((1, 1), jnp.float32)
    udot = _matvec(user_table.T, wf[:, 0:50], zero)
    mdot = _matvec(movie_table.T, wf[:, 50:100], wf[:, 100:101])
    uidx = user.astype(jnp.int32).reshape(NW, BPW)
    midx = movie.astype(jnp.int32).reshape(NW, BPW)
    out = _sc_kernel(uidx, midx, udot, mdot)
    return out.reshape(BATCH, 1)


# speedup vs baseline: 7.5219x; 1.3843x over previous
import functools

import jax
import jax.numpy as jnp
from jax import lax
from jax.experimental import pallas as pl
from jax.experimental.pallas import tpu as pltpu
from jax.experimental.pallas import tpu_sc as plsc

NC = 2
NS = 16
NW = NC * NS
BATCH = 16384
BPW = BATCH // NW
NCHUNK = 4
CHUNK = BPW // NCHUNK
BLK = 4096


def _collapse_body(W1r, b1r, W2r, b2r, W3r, b3r, outr):
    w32 = jnp.dot(W3r[...], W2r[...], preferred_element_type=jnp.float32,
                  precision=lax.Precision.HIGHEST)
    w100 = jnp.dot(w32, W1r[...], preferred_element_type=jnp.float32,
                   precision=lax.Precision.HIGHEST)
    c = jnp.sum(w32 * b1r[...]) + jnp.sum(W3r[...] * b2r[...]) + b3r[0, 0]
    outr[...] = jnp.concatenate(
        [w100, jnp.reshape(c, (1, 1)), jnp.zeros((1, 27), jnp.float32)], axis=1)


def _collapse(W1, b1, W2, b2, W3, b3):
    return pl.pallas_call(
        _collapse_body,
        out_shape=jax.ShapeDtypeStruct((1, 128), jnp.float32),
    )(W1, b1.reshape(1, 40), W2, b2.reshape(1, 20), W3, b3.reshape(1, 1))


def _matvec_body(w_ref, tab_ref, bias_ref, out_ref):
    prod = tab_ref[...] * w_ref[...]
    out_ref[...] = jnp.sum(prod, axis=0) + bias_ref[0, 0]


def _matvec(tab_t, w, bias):
    n = tab_t.shape[1]
    grid = (n + BLK - 1) // BLK
    out = pl.pallas_call(
        _matvec_body,
        grid=(grid,),
        in_specs=[
            pl.BlockSpec((50, 1), lambda i: (0, 0)),
            pl.BlockSpec((50, BLK), lambda i: (0, i)),
            pl.BlockSpec((1, 1), lambda i: (0, 0)),
        ],
        out_specs=pl.BlockSpec((BLK,), lambda i: (i,)),
        out_shape=jax.ShapeDtypeStruct((n,), jnp.float32),
    )(w, tab_t, bias)
    return out


def _sc_body(uidx_hbm, midx_hbm, udot_hbm, mdot_hbm, out_hbm,
             uidx_v, midx_v, uval_v, mval_v, sem):
    wid = lax.axis_index("s") * NC + lax.axis_index("c")
    for j in range(NCHUNK):
        pltpu.sync_copy(uidx_hbm.at[wid, pl.ds(j * CHUNK, CHUNK)], uidx_v.at[j])
        pltpu.sync_copy(midx_hbm.at[wid, pl.ds(j * CHUNK, CHUNK)], midx_v.at[j])
    copies = []
    for j in range(NCHUNK):
        copies.append(pltpu.async_copy(udot_hbm.at[uidx_v.at[j]], uval_v.at[j], sem))
        copies.append(pltpu.async_copy(mdot_hbm.at[midx_v.at[j]], mval_v.at[j], sem))
    for cp in copies:
        cp.wait()
    for j in range(NCHUNK):
        for v in range(CHUNK // 16):
            s = (uval_v[j, pl.ds(v * 16, 16)] + mval_v[j, pl.ds(v * 16, 16)])
            uval_v[j, pl.ds(v * 16, 16)] = s
        pltpu.sync_copy(uval_v.at[j], out_hbm.at[wid, pl.ds(j * CHUNK, CHUNK)])


_sc_kernel = functools.partial(
    pl.kernel,
    out_type=jax.ShapeDtypeStruct((NW, BPW), jnp.float32),
    mesh=plsc.VectorSubcoreMesh(core_axis_name="c", subcore_axis_name="s",
                                num_cores=NC, num_subcores=NS),
    compiler_params=pltpu.CompilerParams(needs_layout_passes=False,
                                         use_tc_tiling_on_sc=False),
    scratch_types=[
        pltpu.VMEM((NCHUNK, CHUNK), jnp.int32),
        pltpu.VMEM((NCHUNK, CHUNK), jnp.int32),
        pltpu.VMEM((NCHUNK, CHUNK), jnp.float32),
        pltpu.VMEM((NCHUNK, CHUNK), jnp.float32),
        pltpu.SemaphoreType.DMA,
    ],
)(_sc_body)


def kernel(user, movie, user_table, movie_table, W1, b1, W2, b2, W3, b3):
    wf = _collapse(W1, b1, W2, b2, W3, b3)
    zero = jnp.zeros((1, 1), jnp.float32)
    udot = _matvec(user_table.T, wf[0, 0:50].reshape(50, 1), zero)
    mdot = _matvec(movie_table.T, wf[0, 50:100].reshape(50, 1), wf[:, 100:101])
    uidx = user.astype(jnp.int32).reshape(NW, BPW)
    midx = movie.astype(jnp.int32).reshape(NW, BPW)
    out = _sc_kernel(uidx, midx, udot, mdot)
    return out.reshape(BATCH, 1)


# speedup vs baseline: 13.1998x; 1.7549x over previous
import functools

import jax
import jax.numpy as jnp
from jax import lax
from jax.experimental import pallas as pl
from jax.experimental.pallas import tpu as pltpu
from jax.experimental.pallas import tpu_sc as plsc

NC = 2
NS = 16
NW = NC * NS
BATCH = 16384
BPW = BATCH // NW
NCHUNK = 4
CHUNK = BPW // NCHUNK
BLK = 16384


def _collapse_body(W1r, b1r, W2r, b2r, W3r, b3r, outr):
    w32 = jnp.dot(W3r[...], W2r[...], preferred_element_type=jnp.float32,
                  precision=lax.Precision.HIGHEST)
    w100 = jnp.dot(w32, W1r[...], preferred_element_type=jnp.float32,
                   precision=lax.Precision.HIGHEST)
    c = jnp.sum(w32 * b1r[...]) + jnp.sum(W3r[...] * b2r[...]) + b3r[0, 0]
    outr[...] = jnp.concatenate(
        [w100, jnp.reshape(c, (1, 1)), jnp.zeros((1, 27), jnp.float32)], axis=1)


def _collapse(W1, b1, W2, b2, W3, b3):
    return pl.pallas_call(
        _collapse_body,
        out_shape=jax.ShapeDtypeStruct((1, 128), jnp.float32),
    )(W1, b1.reshape(1, 40), W2, b2.reshape(1, 20), W3, b3.reshape(1, 1))


def _matvec_body(w_ref, tab_ref, bias_ref, out_ref):
    prod = tab_ref[...] * w_ref[...]
    out_ref[...] = jnp.sum(prod, axis=0) + bias_ref[0, 0]


def _matvec(tab_t, w, bias):
    n = tab_t.shape[1]
    grid = (n + BLK - 1) // BLK
    out = pl.pallas_call(
        _matvec_body,
        grid=(grid,),
        in_specs=[
            pl.BlockSpec((50, 1), lambda i: (0, 0)),
            pl.BlockSpec((50, BLK), lambda i: (0, i)),
            pl.BlockSpec((1, 1), lambda i: (0, 0)),
        ],
        out_specs=pl.BlockSpec((BLK,), lambda i: (i,)),
        out_shape=jax.ShapeDtypeStruct((n,), jnp.float32),
    )(w, tab_t, bias)
    return out


def _sc_body(uidx_hbm, midx_hbm, udot_hbm, mdot_hbm, out_hbm,
             uidx_v, midx_v, uval_v, mval_v, sem):
    wid = lax.axis_index("s") * NC + lax.axis_index("c")
    for j in range(NCHUNK):
        pltpu.sync_copy(uidx_hbm.at[wid, pl.ds(j * CHUNK, CHUNK)], uidx_v.at[j])
        pltpu.sync_copy(midx_hbm.at[wid, pl.ds(j * CHUNK, CHUNK)], midx_v.at[j])
    copies = []
    for j in range(NCHUNK):
        copies.append(pltpu.async_copy(udot_hbm.at[uidx_v.at[j]], uval_v.at[j], sem))
        copies.append(pltpu.async_copy(mdot_hbm.at[midx_v.at[j]], mval_v.at[j], sem))
    for cp in copies:
        cp.wait()
    for j in range(NCHUNK):
        for v in range(CHUNK // 16):
            s = (uval_v[j, pl.ds(v * 16, 16)] + mval_v[j, pl.ds(v * 16, 16)])
            uval_v[j, pl.ds(v * 16, 16)] = s
        pltpu.sync_copy(uval_v.at[j], out_hbm.at[wid, pl.ds(j * CHUNK, CHUNK)])


_sc_kernel = functools.partial(
    pl.kernel,
    out_type=jax.ShapeDtypeStruct((NW, BPW), jnp.float32),
    mesh=plsc.VectorSubcoreMesh(core_axis_name="c", subcore_axis_name="s",
                                num_cores=NC, num_subcores=NS),
    compiler_params=pltpu.CompilerParams(needs_layout_passes=False,
                                         use_tc_tiling_on_sc=False),
    scratch_types=[
        pltpu.VMEM((NCHUNK, CHUNK), jnp.int32),
        pltpu.VMEM((NCHUNK, CHUNK), jnp.int32),
        pltpu.VMEM((NCHUNK, CHUNK), jnp.float32),
        pltpu.VMEM((NCHUNK, CHUNK), jnp.float32),
        pltpu.SemaphoreType.DMA,
    ],
)(_sc_body)


def kernel(user, movie, user_table, movie_table, W1, b1, W2, b2, W3, b3):
    wf = _collapse(W1, b1, W2, b2, W3, b3)
    zero = jnp.zeros((1, 1), jnp.float32)
    udot = _matvec(user_table.T, wf[0, 0:50].reshape(50, 1), zero)
    mdot = _matvec(movie_table.T, wf[0, 50:100].reshape(50, 1), wf[:, 100:101])
    uidx = user.astype(jnp.int32).reshape(NW, BPW)
    midx = movie.astype(jnp.int32).reshape(NW, BPW)
    out = _sc_kernel(uidx, midx, udot, mdot)
    return out.reshape(BATCH, 1)


# speedup vs baseline: 14.9352x; 1.1315x over previous
import functools

import jax
import jax.numpy as jnp
from jax import lax
from jax.experimental import pallas as pl
from jax.experimental.pallas import tpu as pltpu
from jax.experimental.pallas import tpu_sc as plsc

NC = 2
NS = 16
NW = NC * NS
BATCH = 16384
BPW = BATCH // NW
NCHUNK = 4
CHUNK = BPW // NCHUNK
BLK = 65536


def _collapse_body(W1r, b1r, W2r, b2r, W3r, b3r, outr):
    w32 = jnp.dot(W3r[...], W2r[...], preferred_element_type=jnp.float32,
                  precision=lax.Precision.HIGHEST)
    w100 = jnp.dot(w32, W1r[...], preferred_element_type=jnp.float32,
                   precision=lax.Precision.HIGHEST)
    c = jnp.sum(w32 * b1r[...]) + jnp.sum(W3r[...] * b2r[...]) + b3r[0, 0]
    outr[...] = jnp.concatenate(
        [w100, jnp.reshape(c, (1, 1)), jnp.zeros((1, 27), jnp.float32)], axis=1)


def _collapse(W1, b1, W2, b2, W3, b3):
    return pl.pallas_call(
        _collapse_body,
        out_shape=jax.ShapeDtypeStruct((1, 128), jnp.float32),
    )(W1, b1.reshape(1, 40), W2, b2.reshape(1, 20), W3, b3.reshape(1, 1))


def _matvec_body(w_ref, tab_ref, bias_ref, out_ref):
    prod = tab_ref[...] * w_ref[...]
    out_ref[...] = jnp.sum(prod, axis=0) + bias_ref[0, 0]


def _matvec(tab_t, w, bias):
    n = tab_t.shape[1]
    grid = (n + BLK - 1) // BLK
    out = pl.pallas_call(
        _matvec_body,
        grid=(grid,),
        in_specs=[
            pl.BlockSpec((50, 1), lambda i: (0, 0)),
            pl.BlockSpec((50, BLK), lambda i: (0, i)),
            pl.BlockSpec((1, 1), lambda i: (0, 0)),
        ],
        out_specs=pl.BlockSpec((BLK,), lambda i: (i,)),
        out_shape=jax.ShapeDtypeStruct((n,), jnp.float32),
    )(w, tab_t, bias)
    return out


def _sc_body(uidx_hbm, midx_hbm, udot_hbm, mdot_hbm, out_hbm,
             uidx_v, midx_v, uval_v, mval_v, sem):
    wid = lax.axis_index("s") * NC + lax.axis_index("c")
    for j in range(NCHUNK):
        pltpu.sync_copy(uidx_hbm.at[wid, pl.ds(j * CHUNK, CHUNK)], uidx_v.at[j])
        pltpu.sync_copy(midx_hbm.at[wid, pl.ds(j * CHUNK, CHUNK)], midx_v.at[j])
    copies = []
    for j in range(NCHUNK):
        copies.append(pltpu.async_copy(udot_hbm.at[uidx_v.at[j]], uval_v.at[j], sem))
        copies.append(pltpu.async_copy(mdot_hbm.at[midx_v.at[j]], mval_v.at[j], sem))
    for cp in copies:
        cp.wait()
    for j in range(NCHUNK):
        for v in range(CHUNK // 16):
            s = (uval_v[j, pl.ds(v * 16, 16)] + mval_v[j, pl.ds(v * 16, 16)])
            uval_v[j, pl.ds(v * 16, 16)] = s
        pltpu.sync_copy(uval_v.at[j], out_hbm.at[wid, pl.ds(j * CHUNK, CHUNK)])


_sc_kernel = functools.partial(
    pl.kernel,
    out_type=jax.ShapeDtypeStruct((NW, BPW), jnp.float32),
    mesh=plsc.VectorSubcoreMesh(core_axis_name="c", subcore_axis_name="s",
                                num_cores=NC, num_subcores=NS),
    compiler_params=pltpu.CompilerParams(needs_layout_passes=False,
                                         use_tc_tiling_on_sc=False),
    scratch_types=[
        pltpu.VMEM((NCHUNK, CHUNK), jnp.int32),
        pltpu.VMEM((NCHUNK, CHUNK), jnp.int32),
        pltpu.VMEM((NCHUNK, CHUNK), jnp.float32),
        pltpu.VMEM((NCHUNK, CHUNK), jnp.float32),
        pltpu.SemaphoreType.DMA,
    ],
)(_sc_body)


def kernel(user, movie, user_table, movie_table, W1, b1, W2, b2, W3, b3):
    wf = _collapse(W1, b1, W2, b2, W3, b3)
    zero = jnp.zeros((1, 1), jnp.float32)
    udot = _matvec(user_table.T, wf[0, 0:50].reshape(50, 1), zero)
    mdot = _matvec(movie_table.T, wf[0, 50:100].reshape(50, 1), wf[:, 100:101])
    uidx = user.astype(jnp.int32).reshape(NW, BPW)
    midx = movie.astype(jnp.int32).reshape(NW, BPW)
    out = _sc_kernel(uidx, midx, udot, mdot)
    return out.reshape(BATCH, 1)
